# Initial kernel scaffold; baseline (speedup 1.0000x reference)
import functools

import jax
import jax.numpy as jnp
from jax import lax
from jax.experimental import pallas as pl
from jax.experimental.pallas import tpu as pltpu
from jax.experimental.pallas import tpu_sc as plsc

VOCAB = 1000
DIM = 64
BATCH = 4096

NC = 2
NS = 16
NW = NC * NS
LANES = 16
B_PER_W = BATCH // NW
GROUPS = B_PER_W // LANES
CHUNKS = DIM // LANES


def _sc_body(ctx_idx_hbm, tgt_idx_hbm, cxt_hbm, emb_hbm, out_hbm,
             cidx_v, tidx_v, crows_v, trows_v, pbuf_v, out_v,
             sem_c, sem_t):
    wid = lax.axis_index("s") * NC + lax.axis_index("c")
    base = wid * B_PER_W

    pltpu.sync_copy(ctx_idx_hbm.at[pl.ds(base, B_PER_W)], cidx_v)
    pltpu.sync_copy(tgt_idx_hbm.at[pl.ds(base, B_PER_W)], tidx_v)
    cp_c = pltpu.async_copy(cxt_hbm.at[cidx_v], crows_v, sem_c)
    cp_t = pltpu.async_copy(emb_hbm.at[tidx_v], trows_v, sem_t)
    cp_c.wait()
    cp_t.wait()

    row_ids = lax.iota(jnp.int32, LANES)
    for g in range(GROUPS):
        for j in range(LANES):
            r = g * LANES + j
            p = crows_v[r, pl.ds(0, LANES)] * trows_v[r, pl.ds(0, LANES)]
            for k in range(1, CHUNKS):
                p = p + (crows_v[r, pl.ds(k * LANES, LANES)]
                         * trows_v[r, pl.ds(k * LANES, LANES)])
            pbuf_v[j, pl.ds(0, LANES)] = p
        acc = plsc.load_gather(
            pbuf_v, [row_ids, jnp.zeros((LANES,), jnp.int32)])
        for l in range(1, LANES):
            col = plsc.load_gather(
                pbuf_v, [row_ids, jnp.full((LANES,), l, jnp.int32)])
            acc = acc + col
        out_v[pl.ds(g * LANES, LANES)] = acc

    pltpu.sync_copy(out_v, out_hbm.at[pl.ds(base, B_PER_W)])


@jax.jit
def _sc_call(context_idxs, target_idxs, cxt_table, emb_table):
    mesh = plsc.VectorSubcoreMesh(core_axis_name="c", subcore_axis_name="s")
    kern = pl.kernel(
        _sc_body,
        mesh=mesh,
        out_type=jax.ShapeDtypeStruct((BATCH,), jnp.float32),
        scratch_types=[
            pltpu.VMEM((B_PER_W,), jnp.int32),
            pltpu.VMEM((B_PER_W,), jnp.int32),
            pltpu.VMEM((B_PER_W, DIM), jnp.float32),
            pltpu.VMEM((B_PER_W, DIM), jnp.float32),
            pltpu.VMEM((LANES, LANES + 1), jnp.float32),
            pltpu.VMEM((B_PER_W,), jnp.float32),
            pltpu.SemaphoreType.DMA,
            pltpu.SemaphoreType.DMA,
        ],
    )
    return kern(context_idxs, target_idxs, cxt_table, emb_table)


def kernel(context_idxs, target_idxs, cxt_table, emb_table):
    out = _sc_call(context_idxs.astype(jnp.int32),
                   target_idxs.astype(jnp.int32),
                   cxt_table, emb_table)
    return out.reshape(-1, 1)


# speedup vs baseline: 2.6285x; 2.6285x over previous
import functools

import jax
import jax.numpy as jnp
from jax import lax
from jax.experimental import pallas as pl
from jax.experimental.pallas import tpu as pltpu
from jax.experimental.pallas import tpu_sc as plsc

VOCAB = 1000
DIM = 64
BATCH = 4096

NC = 2
NS = 16
NW = NC * NS
LANES = 16
B_PER_W = BATCH // NW
GROUPS = B_PER_W // LANES
CHUNKS = DIM // LANES


_GATHER_DNUMS = lax.GatherDimensionNumbers(
    offset_dims=(), collapsed_slice_dims=(0,), start_index_map=(0,))


def _lane_perm(x, idx):
    return lax.gather(
        x, idx[:, None], _GATHER_DNUMS, slice_sizes=(1,),
        mode=lax.GatherScatterMode.PROMISE_IN_BOUNDS)


def _sc_body(ctx_idx_hbm, tgt_idx_hbm, cxt_hbm, emb_hbm, out_hbm,
             cidx_v, tidx_v, crows_v, trows_v, out_v,
             sem_c, sem_t):
    wid = lax.axis_index("s") * NC + lax.axis_index("c")
    base = wid * B_PER_W

    pltpu.sync_copy(ctx_idx_hbm.at[pl.ds(base, B_PER_W)], cidx_v)
    pltpu.sync_copy(tgt_idx_hbm.at[pl.ds(base, B_PER_W)], tidx_v)
    cp_c = pltpu.async_copy(cxt_hbm.at[cidx_v], crows_v, sem_c)
    cp_t = pltpu.async_copy(emb_hbm.at[tidx_v], trows_v, sem_t)
    cp_c.wait()
    cp_t.wait()

    row_ids = lax.iota(jnp.int32, LANES)
    for g in range(GROUPS):
        acc = jnp.zeros((LANES,), jnp.float32)
        for j in range(LANES):
            r = g * LANES + j
            p = crows_v[r, pl.ds(0, LANES)] * trows_v[r, pl.ds(0, LANES)]
            for k in range(1, CHUNKS):
                p = p + (crows_v[r, pl.ds(k * LANES, LANES)]
                         * trows_v[r, pl.ds(k * LANES, LANES)])
            for sh in (8, 4, 2, 1):
                p = p + _lane_perm(p, row_ids ^ sh)
            acc = jnp.where(row_ids == j, p, acc)
        out_v[pl.ds(g * LANES, LANES)] = acc

    pltpu.sync_copy(out_v, out_hbm.at[pl.ds(base, B_PER_W)])


@jax.jit
def _sc_call(context_idxs, target_idxs, cxt_table, emb_table):
    mesh = plsc.VectorSubcoreMesh(core_axis_name="c", subcore_axis_name="s")
    kern = pl.kernel(
        _sc_body,
        mesh=mesh,
        compiler_params=pltpu.CompilerParams(use_tc_tiling_on_sc=False),
        out_type=jax.ShapeDtypeStruct((BATCH,), jnp.float32),
        scratch_types=[
            pltpu.VMEM((B_PER_W,), jnp.int32),
            pltpu.VMEM((B_PER_W,), jnp.int32),
            pltpu.VMEM((B_PER_W, DIM), jnp.float32),
            pltpu.VMEM((B_PER_W, DIM), jnp.float32),
            pltpu.VMEM((B_PER_W,), jnp.float32),
            pltpu.SemaphoreType.DMA,
            pltpu.SemaphoreType.DMA,
        ],
    )
    return kern(context_idxs, target_idxs, cxt_table, emb_table)


def kernel(context_idxs, target_idxs, cxt_table, emb_table):
    out = _sc_call(context_idxs.astype(jnp.int32),
                   target_idxs.astype(jnp.int32),
                   cxt_table, emb_table)
    return out.reshape(-1, 1)
